# Initial kernel scaffold; baseline (speedup 1.0000x reference)
import jax
import jax.numpy as jnp
from jax.experimental import pallas as pl


def kernel(z, codebook):
    raise NotImplementedError("write your pallas kernel here")


# speedup vs baseline: 1.3084x; 1.3084x over previous
import functools

import jax
import jax.numpy as jnp
from jax import lax
from jax.experimental import pallas as pl
from jax.experimental.pallas import tpu as pltpu
from jax.experimental.pallas import tpu_sc as plsc

T_BLK = 256
WINDOW = 2736

_NUM_CORES = 2
_NUM_SUBCORES = 16
_NUM_WORKERS = _NUM_CORES * _NUM_SUBCORES
_GATHER_CHUNK = 128


def _bf16_rne(v):
    bits = lax.bitcast_convert_type(v, jnp.int32)
    r = (bits + jnp.int32(0x7FFF) + ((bits >> 16) & 1)) & jnp.int32(-0x10000)
    return lax.bitcast_convert_type(r, jnp.float32)


def _dist_argmin_body(a_ref, z_ref, c_ref, c2_ref, idx_ref):
    k_total = c_ref.shape[0]
    mm = lax.dot_general(
        c_ref[...], z_ref[0],
        (((1,), (0,)), ((), ())),
        preferred_element_type=jnp.float32,
    )
    a = a_ref[0]
    c2 = c2_ref[...]
    d = (a - 2.0 * mm) + c2
    kidx = lax.broadcasted_iota(jnp.int32, d.shape, 0)

    win_v, win_i = [], []
    lo = 0
    while lo < k_total:
        hi = min(lo + WINDOW, k_total)
        seg = d[lo:hi, :]
        wmin = jnp.min(seg, axis=0, keepdims=True)
        warg = jnp.min(
            jnp.where(seg == wmin, kidx[lo:hi, :], 2**30), axis=0, keepdims=True
        )
        win_v.append(wmin)
        win_i.append(warg)
        lo = hi

    av = _bf16_rne(win_v[0])
    ai = win_i[0]
    for w in range(1, len(win_v)):
        ai = jnp.where(av <= win_v[w], ai, win_i[w])
        av = _bf16_rne(jnp.minimum(av, win_v[w]))
    idx_ref[...] = ai[None]


def _nearest_indices(a3, z_bf, cb_bf, c2c):
    b, d, t = z_bf.shape
    k = cb_bf.shape[0]
    grid = (b, t // T_BLK)
    return pl.pallas_call(
        _dist_argmin_body,
        grid=grid,
        in_specs=[
            pl.BlockSpec((1, 1, T_BLK), lambda i, j: (i, 0, j)),
            pl.BlockSpec((1, d, T_BLK), lambda i, j: (i, 0, j)),
            pl.BlockSpec((k, d), lambda i, j: (0, 0)),
            pl.BlockSpec((k, 1), lambda i, j: (0, 0)),
        ],
        out_specs=pl.BlockSpec((1, 1, T_BLK), lambda i, j: (i, 0, j)),
        out_shape=jax.ShapeDtypeStruct((b, 1, t), jnp.int32),
    )(a3, z_bf, cb_bf, c2c)


def _sc_gather(codebook, idx3):
    nw, n_chunks, _ = idx3.shape
    bt = nw * n_chunks * _GATHER_CHUNK
    d = codebook.shape[1]
    b_per_w = bt // _NUM_WORKERS
    mesh = plsc.VectorSubcoreMesh(core_axis_name="c", subcore_axis_name="s")

    @functools.partial(
        pl.kernel,
        mesh=mesh,
        out_type=jax.ShapeDtypeStruct((bt, d), jnp.float32),
        scratch_types=[
            pltpu.VMEM((n_chunks, _GATHER_CHUNK), jnp.int32),
            pltpu.VMEM((_GATHER_CHUNK, d), jnp.float32),
            pltpu.SemaphoreType.DMA,
        ],
    )
    def gather_kernel(table_hbm, idx_hbm, out_hbm, idx_v, rows_v, sem):
        wid = lax.axis_index("s") * _NUM_CORES + lax.axis_index("c")
        base = wid * b_per_w
        pltpu.sync_copy(idx_hbm.at[wid], idx_v)
        for c in range(n_chunks):
            pltpu.async_copy(table_hbm.at[idx_v.at[c]], rows_v, sem).wait()
            pltpu.sync_copy(
                rows_v, out_hbm.at[pl.ds(base + c * _GATHER_CHUNK, _GATHER_CHUNK)]
            )

    return gather_kernel(codebook, idx3)


def kernel(z, codebook):
    b, d, t = z.shape
    flat_z = jnp.transpose(z, (0, 2, 1)).reshape(-1, d)
    a = jnp.sum(flat_z * flat_z, axis=1)
    c2 = jnp.sum(codebook * codebook, axis=1)
    idx = _nearest_indices(
        a.reshape(b, 1, t),
        z.astype(jnp.bfloat16),
        codebook.astype(jnp.bfloat16),
        c2[:, None],
    )
    idx3 = idx.reshape(_NUM_WORKERS, -1, _GATHER_CHUNK)
    quantized = _sc_gather(codebook, idx3)
    return jnp.transpose(quantized.reshape(b, t, d), (0, 2, 1))
